# Initial kernel scaffold; baseline (speedup 1.0000x reference)
import jax
import jax.numpy as jnp
from jax.experimental import pallas as pl


def kernel(sequence, table):
    raise NotImplementedError("write your pallas kernel here")


# speedup vs baseline: 5.1887x; 5.1887x over previous
import functools

import jax
import jax.numpy as jnp
from jax import lax
from jax.experimental import pallas as pl
from jax.experimental.pallas import tpu as pltpu
from jax.experimental.pallas import tpu_sc as plsc

NC, NS = 2, 16
NW = NC * NS
C = 128
G = 4
CHUNK = G * C
PV = 32


def kernel(sequence, table):
    B, S = sequence.shape
    V, D = table.shape
    n = B * S
    n2 = n // 2
    per_w = n2 // NW
    steps = per_w // CHUNK
    assert steps * CHUNK * NW == n2
    seq_flat = sequence.reshape(n).astype(jnp.int32)

    tpad = jnp.pad(table, ((0, PV - V), (0, 0)))
    left = jnp.broadcast_to(table[:, None, :], (V, PV, D))
    right = jnp.broadcast_to(tpad[None, :, :], (V, PV, D))
    pairtab = jnp.concatenate([left, right], axis=-1).reshape(V * PV, 2 * D)

    mesh = plsc.VectorSubcoreMesh(core_axis_name="c", subcore_axis_name="s")

    @functools.partial(
        pl.kernel,
        out_type=jax.ShapeDtypeStruct((n2, 2 * D), jnp.float32),
        mesh=mesh,
        scratch_types=[
            pltpu.VMEM((2 * CHUNK,), jnp.int32),
            pltpu.VMEM((G, C), jnp.int32),
            pltpu.VMEM((CHUNK, 2 * D), jnp.float32),
            pltpu.VMEM_SHARED((V * PV, 2 * D), jnp.float32),
            pltpu.SemaphoreType.DMA,
        ],
        compiler_params=pltpu.CompilerParams(needs_layout_passes=False),
    )
    def emb(seq_hbm, pt_hbm, out_hbm, idx_v, pidx_v, rows_v, pt_sh, sem):
        cid = lax.axis_index("c")
        sid = lax.axis_index("s")
        wid = sid * NC + cid
        idx_base = wid * per_w * 2
        out_base = wid * per_w

        @pl.when(sid == 0)
        def _stage_table():
            pltpu.sync_copy(pt_hbm, pt_sh)

        plsc.subcore_barrier()

        iota16 = lax.iota(jnp.int32, 16)

        @pl.loop(0, steps)
        def step(i):
            off = pl.multiple_of(idx_base + i * 2 * CHUNK, 8)
            pltpu.sync_copy(seq_hbm.at[pl.ds(off, 2 * CHUNK)], idx_v)
            for t in range(CHUNK // 16):
                col_e = t * 32 + 2 * iota16
                a = plsc.load_gather(idx_v, [col_e])
                b = plsc.load_gather(idx_v, [col_e + 1])
                pidx_v[t // 8, pl.ds((t % 8) * 16, 16)] = a * PV + b
            copies = [
                pltpu.async_copy(
                    pt_sh.at[pidx_v.at[g]],
                    rows_v.at[pl.ds(g * C, C)],
                    sem,
                )
                for g in range(G)
            ]
            for cp in copies:
                cp.wait()
            pltpu.sync_copy(rows_v, out_hbm.at[pl.ds(out_base + i * CHUNK, CHUNK)])

    out = emb(seq_flat, pairtab)
    return out.reshape(B, S, D)


# speedup vs baseline: 5.8209x; 1.1218x over previous
import functools

import jax
import jax.numpy as jnp
from jax import lax
from jax.experimental import pallas as pl
from jax.experimental.pallas import tpu as pltpu
from jax.experimental.pallas import tpu_sc as plsc

NC, NS = 2, 16
NW = NC * NS
C = 128
G = 2
CHUNK = G * C
PV = 32


def kernel(sequence, table):
    B, S = sequence.shape
    V, D = table.shape
    n = B * S
    n2 = n // 2
    per_w = n2 // NW
    steps = per_w // CHUNK
    assert steps * CHUNK * NW == n2 and steps % 2 == 0

    seq_flat = sequence.reshape(n).astype(jnp.int32)

    tpad = jnp.pad(table, ((0, PV - V), (0, 0)))
    left = jnp.broadcast_to(table[:, None, :], (V, PV, D))
    right = jnp.broadcast_to(tpad[None, :, :], (V, PV, D))
    pairtab = jnp.concatenate([left, right], axis=-1).reshape(V * PV, 2 * D)

    mesh = plsc.VectorSubcoreMesh(core_axis_name="c", subcore_axis_name="s")

    @functools.partial(
        pl.kernel,
        out_type=jax.ShapeDtypeStruct((n2, 2 * D), jnp.float32),
        mesh=mesh,
        scratch_types=[
            pltpu.VMEM((2 * CHUNK,), jnp.int32),
            pltpu.VMEM((2 * CHUNK,), jnp.int32),
            pltpu.VMEM((G, C), jnp.int32),
            pltpu.VMEM((G, C), jnp.int32),
            pltpu.VMEM((CHUNK, 2 * D), jnp.float32),
            pltpu.VMEM((CHUNK, 2 * D), jnp.float32),
            pltpu.VMEM_SHARED((V * PV, 2 * D), jnp.float32),
            pltpu.SemaphoreType.DMA,
            pltpu.SemaphoreType.DMA,
            pltpu.SemaphoreType.DMA,
            pltpu.SemaphoreType.DMA,
        ],
        compiler_params=pltpu.CompilerParams(needs_layout_passes=False),
    )
    def emb(seq_hbm, pt_hbm, out_hbm,
            idx0_v, idx1_v, pidx0_v, pidx1_v, rows0_v, rows1_v,
            pt_sh, gsem0, gsem1, ssem0, ssem1):
        cid = lax.axis_index("c")
        sid = lax.axis_index("s")
        wid = sid * NC + cid
        idx_base = wid * per_w * 2
        out_base = wid * per_w

        idx_refs = (idx0_v, idx1_v)
        pidx_refs = (pidx0_v, pidx1_v)
        rows_refs = (rows0_v, rows1_v)
        gsems = (gsem0, gsem1)
        ssems = (ssem0, ssem1)

        @pl.when(sid == 0)
        def _stage_table():
            pltpu.sync_copy(pt_hbm, pt_sh)

        plsc.subcore_barrier()

        iota16 = lax.iota(jnp.int32, 16)

        def load_idx(k, idx_ref):
            off = pl.multiple_of(idx_base + k * 2 * CHUNK, 8)
            pltpu.sync_copy(seq_hbm.at[pl.ds(off, 2 * CHUNK)], idx_ref)

        def compute_pidx(idx_ref, pidx_ref):
            for t in range(CHUNK // 16):
                col_e = t * 32 + 2 * iota16
                a = plsc.load_gather(idx_ref, [col_e])
                b = plsc.load_gather(idx_ref, [col_e + 1])
                pidx_ref[t // 8, pl.ds((t % 8) * 16, 16)] = a * PV + b

        def gather_copies(p):
            return [
                pltpu.make_async_copy(
                    pt_sh.at[pidx_refs[p].at[g]],
                    rows_refs[p].at[pl.ds(g * C, C)],
                    gsems[p],
                )
                for g in range(G)
            ]

        def store_copy(k, p):
            return pltpu.make_async_copy(
                rows_refs[p],
                out_hbm.at[pl.ds(out_base + k * CHUNK, CHUNK)],
                ssems[p],
            )

        @pl.loop(0, steps // 2)
        def jloop(j):
            for p in range(2):
                k = 2 * j + p
                load_idx(k, idx_refs[p])
                compute_pidx(idx_refs[p], pidx_refs[p])

                @pl.when(j >= 1)
                def _wait_store(k=k, p=p):
                    store_copy(k - 2, p).wait()

                for cp in gather_copies(p):
                    cp.start()

                def _drain_prev(k=k, p=p):
                    for cp in gather_copies(1 - p):
                        cp.wait()
                    store_copy(k - 1, 1 - p).start()

                if p == 1:
                    _drain_prev()
                else:
                    pl.when(j >= 1)(_drain_prev)

        for cp in gather_copies(1):
            cp.wait()
        store_copy(steps - 1, 1).start()
        store_copy(steps - 2, 0).wait()
        store_copy(steps - 1, 1).wait()

    out = emb(seq_flat, pairtab)
    return out.reshape(B, S, D)
